# baseline (speedup 1.0000x reference)
import functools

import jax
import jax.numpy as jnp
from jax import lax
from jax.experimental import pallas as pl
from jax.experimental.pallas import tpu as pltpu
from jax.experimental.pallas import tpu_sc as plsc

N = 819200
L = 6
D = 64
PC = D // 2
ST = PC + 1
ROWS = 2 * 26 + 1000

NC = 2
NS = 16
NW = NC * NS
PER_W = N // NW
CP = 256
NCHUNK = PER_W // CP
NP = NCHUNK // 2
G = CP // 16

TAB_LEN = ROWS * ST
PAIR_LEN = 101 * ST
ZROW = 100 * ST
MASK_HI = -65536


def _unpack2(x):
    lo = plsc.bitcast(lax.shift_left(x, jnp.full_like(x, 16)), jnp.float32)
    hi = plsc.bitcast(lax.bitwise_and(x, jnp.full_like(x, MASK_HI)),
                      jnp.float32)
    return lo, hi


def _sc_body(fc_h, dg_h, md_h, vl_h, tab_h, p0_h, p1_h, p2_h, out_h,
             tabv, p0v, p1v, p2v, fcv, dgv, mdv, vlv, outv,
             ins0, ins1, outs0, outs1):
    c = lax.axis_index("c")
    s = lax.axis_index("s")
    wid = s * NC + c
    base0 = wid * PER_W

    pltpu.sync_copy(tab_h, tabv)
    pltpu.sync_copy(p0_h, p0v)
    pltpu.sync_copy(p1_h, p1v)
    pltpu.sync_copy(p2_h, p2v)

    iot = lax.iota(jnp.int32, 16)
    insems = (ins0, ins1)
    outsems = (outs0, outs1)

    def in_copies(ch, b):
        gb = base0 + ch * CP
        return (
            (fc_h.at[pl.ds(gb, CP)], fcv.at[pl.ds(b * CP, CP)]),
            (dg_h.at[pl.ds(gb * L, CP * L)], dgv.at[pl.ds(b * CP * L, CP * L)]),
            (md_h.at[pl.ds(gb, CP)], mdv.at[pl.ds(b * CP, CP)]),
            (vl_h.at[pl.ds(gb, CP)], vlv.at[pl.ds(b * CP, CP)]),
        )

    def start_in(ch, b):
        for src, dst in in_copies(ch, b):
            pltpu.async_copy(src, dst, insems[b])

    def drain_in(ch, b):
        for src, dst in in_copies(ch, b):
            pltpu.make_async_copy(src, dst, insems[b]).wait()

    def out_copy(ch, b):
        gb = base0 + ch * CP
        return (outv.at[pl.ds(b * CP * D, CP * D)],
                out_h.at[pl.ds(gb * D, CP * D)])

    def compute(b):
        ob_off = b * CP * D

        def group_body(g, carry2):
            t0 = b * CP + g * 16
            local = g * 16 + iot
            fc16 = fcv[pl.ds(t0, 16)]
            md16 = mdv[pl.ds(t0, 16)]
            vl16 = vlv[pl.ds(t0, 16)]
            t6 = local * L
            dgslice = dgv.at[pl.ds(b * CP * L, CP * L)]
            d0 = plsc.load_gather(dgslice, [t6])
            d1 = plsc.load_gather(dgslice, [t6 + 1])
            d2 = plsc.load_gather(dgslice, [t6 + 2])
            d3 = plsc.load_gather(dgslice, [t6 + 3])
            d4 = plsc.load_gather(dgslice, [t6 + 4])
            d5 = plsc.load_gather(dgslice, [t6 + 5])
            is_lab = md16 == 2
            zr = jnp.full_like(t6, ZROW)
            rb = fc16 * ST
            b01 = jnp.where(is_lab, zr, (d0 * 10 + d1) * ST)
            b23 = jnp.where(is_lab, zr, (d2 * 10 + d3) * ST)
            b45 = jnp.where(is_lab, zr, (d4 * 10 + d5) * ST)
            ob = local * D
            scale = jnp.where(is_lab, vl16, jnp.ones_like(vl16))
            scale2 = plsc.pack(scale, scale, format=plsc.PackFormat.INTERLEAVED)
            outsl = outv.at[pl.ds(ob_off, CP * D)]
            @plsc.parallel_loop(0, PC, unroll=8)
            def col_body(jp):
                jv = (iot + jp) & (PC - 1)
                bc = plsc.load_gather(tabv, [rb + jv])
                w1 = plsc.load_gather(p0v, [b01 + jv])
                w2 = plsc.load_gather(p1v, [b23 + jv])
                w3 = plsc.load_gather(p2v, [b45 + jv])
                bcb = plsc.bitcast(bc, jnp.bfloat16)
                w1b = plsc.bitcast(w1, jnp.bfloat16)
                w2b = plsc.bitcast(w2, jnp.bfloat16)
                w3b = plsc.bitcast(w3, jnp.bfloat16)
                o2 = bcb * scale2 + ((w1b + w2b) + w3b)
                oe, oo = plsc.unpack(o2, format=plsc.PackFormat.INTERLEAVED)
                obe = ob + (jv + jv)
                plsc.store_scatter(outsl, [obe], oe)
                plsc.store_scatter(outsl, [obe + 1], oo)
            return carry2

        lax.fori_loop(0, G, group_body, 0)

    start_in(0, 0)
    start_in(1, 1)

    def pair_body(jp, carry):
        for bbuf in (0, 1):
            ch = 2 * jp + bbuf
            drain_in(ch, bbuf)

            @pl.when(jp >= 1)
            def _wait_out():
                src, dst = out_copy(ch - 2, bbuf)
                pltpu.make_async_copy(src, dst, outsems[bbuf]).wait()

            compute(bbuf)
            src, dst = out_copy(ch, bbuf)
            pltpu.async_copy(src, dst, outsems[bbuf])

            @pl.when(jp < NP - 1)
            def _prefetch():
                start_in(ch + 2, bbuf)
        return carry

    lax.fori_loop(0, NP, pair_body, 0)
    for bbuf in (0, 1):
        src, dst = out_copy(NCHUNK - 2 + bbuf, bbuf)
        pltpu.make_async_copy(src, dst, outsems[bbuf]).wait()


@functools.partial(
    pl.kernel,
    out_type=jax.ShapeDtypeStruct((N * D,), jnp.float32),
    mesh=plsc.VectorSubcoreMesh(core_axis_name="c", subcore_axis_name="s"),
    compiler_params=pltpu.CompilerParams(needs_layout_passes=False),
    scratch_types=[
        pltpu.VMEM((TAB_LEN,), jnp.int32),
        pltpu.VMEM((PAIR_LEN,), jnp.int32),
        pltpu.VMEM((PAIR_LEN,), jnp.int32),
        pltpu.VMEM((PAIR_LEN,), jnp.int32),
        pltpu.VMEM((2 * CP,), jnp.int32),
        pltpu.VMEM((2 * CP * L,), jnp.int32),
        pltpu.VMEM((2 * CP,), jnp.int32),
        pltpu.VMEM((2 * CP,), jnp.float32),
        pltpu.VMEM((2 * CP * D,), jnp.float32),
        pltpu.SemaphoreType.DMA,
        pltpu.SemaphoreType.DMA,
        pltpu.SemaphoreType.DMA,
        pltpu.SemaphoreType.DMA,
    ],
)
def _sc_kernel(fc_h, dg_h, md_h, vl_h, tab_h, p0_h, p1_h, p2_h, out_h,
               tabv, p0v, p1v, p2v, fcv, dgv, mdv, vlv, outv,
               ins0, ins1, outs0, outs1):
    _sc_body(fc_h, dg_h, md_h, vl_h, tab_h, p0_h, p1_h, p2_h, out_h,
             tabv, p0v, p1v, p2v, fcv, dgv, mdv, vlv, outv,
             ins0, ins1, outs0, outs1)


def _pack_rows(t, pad_rows):
    tb = t.astype(jnp.bfloat16)
    u = lax.bitcast_convert_type(tb, jnp.uint16).astype(jnp.uint32)
    lo = u[:, 0::2]
    hi = u[:, 1::2]
    packed = (lo | (hi << 16)).astype(jnp.int32)
    return jnp.pad(packed, ((0, pad_rows), (0, 1))).reshape(-1)


def kernel(first_char, digits, mods, vals, kappa, first_level_w, integer_w):
    pos_w = 1.0 / (jnp.arange(L, dtype=jnp.float32) + 2.0) ** jnp.asarray(
        kappa, jnp.float32)
    pads = []
    for k in range(3):
        pk = (pos_w[2 * k] * integer_w[:, None, :]
              + pos_w[2 * k + 1] * integer_w[None, :, :]).reshape(100, D)
        pads.append(_pack_rows(pk, 1))
    tab = _pack_rows(first_level_w, 0)
    fc = first_char.astype(jnp.int32)
    dg = digits.astype(jnp.int32).reshape(-1)
    md = mods.astype(jnp.int32)
    vl = vals.astype(jnp.float32)
    out = _sc_kernel(fc, dg, md, vl, tab, pads[0], pads[1], pads[2])
    return out.reshape(N, D)


# speedup vs baseline: 1.0546x; 1.0546x over previous
import functools

import jax
import jax.numpy as jnp
from jax import lax
from jax.experimental import pallas as pl
from jax.experimental.pallas import tpu as pltpu
from jax.experimental.pallas import tpu_sc as plsc

N = 819200
L = 6
D = 64
PC = D // 2
ST = PC + 1
ROWS = 2 * 26 + 1000

NC = 2
NS = 16
NW = NC * NS
PER_W = N // NW
CP = 256
NCHUNK = PER_W // CP
NP = NCHUNK // 2
G = CP // 16

TAB_LEN = ROWS * ST
PAIR_LEN = 101 * ST
ZROW = 100 * ST
MASK_HI = -65536


def _unpack2(x):
    lo = plsc.bitcast(lax.shift_left(x, jnp.full_like(x, 16)), jnp.float32)
    hi = plsc.bitcast(lax.bitwise_and(x, jnp.full_like(x, MASK_HI)),
                      jnp.float32)
    return lo, hi


def _sc_body(fc_h, dg_h, md_h, vl_h, tab_h, p0_h, p1_h, p2_h, out_h,
             tabv, p0v, p1v, p2v, fcv, dgv, mdv, vlv, outv,
             ins0, ins1, outs0, outs1):
    c = lax.axis_index("c")
    s = lax.axis_index("s")
    wid = s * NC + c
    base0 = wid * PER_W

    pltpu.sync_copy(tab_h, tabv)
    pltpu.sync_copy(p0_h, p0v)
    pltpu.sync_copy(p1_h, p1v)
    pltpu.sync_copy(p2_h, p2v)

    iot = lax.iota(jnp.int32, 16)
    insems = (ins0, ins1)
    outsems = (outs0, outs1)

    def in_copies(ch, b):
        gb = base0 + ch * CP
        return (
            (fc_h.at[pl.ds(gb, CP)], fcv.at[pl.ds(b * CP, CP)]),
            (dg_h.at[pl.ds(gb * L, CP * L)], dgv.at[pl.ds(b * CP * L, CP * L)]),
            (md_h.at[pl.ds(gb, CP)], mdv.at[pl.ds(b * CP, CP)]),
            (vl_h.at[pl.ds(gb, CP)], vlv.at[pl.ds(b * CP, CP)]),
        )

    def start_in(ch, b):
        for src, dst in in_copies(ch, b):
            pltpu.async_copy(src, dst, insems[b])

    def drain_in(ch, b):
        for src, dst in in_copies(ch, b):
            pltpu.make_async_copy(src, dst, insems[b]).wait()

    def out_copy(ch, b):
        gb = base0 + ch * CP
        return (outv.at[pl.ds(b * CP * D, CP * D)],
                out_h.at[pl.ds(gb * D, CP * D)])

    def compute(b):
        ob_off = b * CP * D

        def group_body(g, carry2):
            t0 = b * CP + g * 16
            local = g * 16 + iot
            fc16 = fcv[pl.ds(t0, 16)]
            md16 = mdv[pl.ds(t0, 16)]
            vl16 = vlv[pl.ds(t0, 16)]
            t6 = local * L
            dgslice = dgv.at[pl.ds(b * CP * L, CP * L)]
            d0 = plsc.load_gather(dgslice, [t6])
            d1 = plsc.load_gather(dgslice, [t6 + 1])
            d2 = plsc.load_gather(dgslice, [t6 + 2])
            d3 = plsc.load_gather(dgslice, [t6 + 3])
            d4 = plsc.load_gather(dgslice, [t6 + 4])
            d5 = plsc.load_gather(dgslice, [t6 + 5])
            is_lab = md16 == 2
            zr = jnp.full_like(t6, ZROW)
            rb = fc16 * ST
            b01 = jnp.where(is_lab, zr, (d0 * 10 + d1) * ST)
            b23 = jnp.where(is_lab, zr, (d2 * 10 + d3) * ST)
            b45 = jnp.where(is_lab, zr, (d4 * 10 + d5) * ST)
            ob = local * D
            scale = jnp.where(is_lab, vl16, jnp.ones_like(vl16))
            scale2 = plsc.pack(scale, scale, format=plsc.PackFormat.INTERLEAVED)
            outsl = outv.at[pl.ds(ob_off, CP * D)]
            @plsc.parallel_loop(0, PC, unroll=2)
            def col_body(jp):
                jv = (iot + jp) & (PC - 1)
                bc = plsc.load_gather(tabv, [rb + jv])
                w1 = plsc.load_gather(p0v, [b01 + jv])
                w2 = plsc.load_gather(p1v, [b23 + jv])
                w3 = plsc.load_gather(p2v, [b45 + jv])
                bcb = plsc.bitcast(bc, jnp.bfloat16)
                w1b = plsc.bitcast(w1, jnp.bfloat16)
                w2b = plsc.bitcast(w2, jnp.bfloat16)
                w3b = plsc.bitcast(w3, jnp.bfloat16)
                o2 = bcb * scale2 + ((w1b + w2b) + w3b)
                oe, oo = plsc.unpack(o2, format=plsc.PackFormat.INTERLEAVED)
                obe = ob + (jv + jv)
                plsc.store_scatter(outsl, [obe], oe)
                plsc.store_scatter(outsl, [obe + 1], oo)
            return carry2

        lax.fori_loop(0, G, group_body, 0)

    start_in(0, 0)
    start_in(1, 1)

    def pair_body(jp, carry):
        for bbuf in (0, 1):
            ch = 2 * jp + bbuf
            drain_in(ch, bbuf)

            @pl.when(jp >= 1)
            def _wait_out():
                src, dst = out_copy(ch - 2, bbuf)
                pltpu.make_async_copy(src, dst, outsems[bbuf]).wait()

            compute(bbuf)
            src, dst = out_copy(ch, bbuf)
            pltpu.async_copy(src, dst, outsems[bbuf])

            @pl.when(jp < NP - 1)
            def _prefetch():
                start_in(ch + 2, bbuf)
        return carry

    lax.fori_loop(0, NP, pair_body, 0)
    for bbuf in (0, 1):
        src, dst = out_copy(NCHUNK - 2 + bbuf, bbuf)
        pltpu.make_async_copy(src, dst, outsems[bbuf]).wait()


@functools.partial(
    pl.kernel,
    out_type=jax.ShapeDtypeStruct((N * D,), jnp.float32),
    mesh=plsc.VectorSubcoreMesh(core_axis_name="c", subcore_axis_name="s"),
    compiler_params=pltpu.CompilerParams(needs_layout_passes=False),
    scratch_types=[
        pltpu.VMEM((TAB_LEN,), jnp.int32),
        pltpu.VMEM((PAIR_LEN,), jnp.int32),
        pltpu.VMEM((PAIR_LEN,), jnp.int32),
        pltpu.VMEM((PAIR_LEN,), jnp.int32),
        pltpu.VMEM((2 * CP,), jnp.int32),
        pltpu.VMEM((2 * CP * L,), jnp.int32),
        pltpu.VMEM((2 * CP,), jnp.int32),
        pltpu.VMEM((2 * CP,), jnp.float32),
        pltpu.VMEM((2 * CP * D,), jnp.float32),
        pltpu.SemaphoreType.DMA,
        pltpu.SemaphoreType.DMA,
        pltpu.SemaphoreType.DMA,
        pltpu.SemaphoreType.DMA,
    ],
)
def _sc_kernel(fc_h, dg_h, md_h, vl_h, tab_h, p0_h, p1_h, p2_h, out_h,
               tabv, p0v, p1v, p2v, fcv, dgv, mdv, vlv, outv,
               ins0, ins1, outs0, outs1):
    _sc_body(fc_h, dg_h, md_h, vl_h, tab_h, p0_h, p1_h, p2_h, out_h,
             tabv, p0v, p1v, p2v, fcv, dgv, mdv, vlv, outv,
             ins0, ins1, outs0, outs1)


def _pack_rows(t, pad_rows):
    tb = t.astype(jnp.bfloat16)
    u = lax.bitcast_convert_type(tb, jnp.uint16).astype(jnp.uint32)
    lo = u[:, 0::2]
    hi = u[:, 1::2]
    packed = (lo | (hi << 16)).astype(jnp.int32)
    return jnp.pad(packed, ((0, pad_rows), (0, 1))).reshape(-1)


def kernel(first_char, digits, mods, vals, kappa, first_level_w, integer_w):
    pos_w = 1.0 / (jnp.arange(L, dtype=jnp.float32) + 2.0) ** jnp.asarray(
        kappa, jnp.float32)
    pads = []
    for k in range(3):
        pk = (pos_w[2 * k] * integer_w[:, None, :]
              + pos_w[2 * k + 1] * integer_w[None, :, :]).reshape(100, D)
        pads.append(_pack_rows(pk, 1))
    tab = _pack_rows(first_level_w, 0)
    fc = first_char.astype(jnp.int32)
    dg = digits.astype(jnp.int32).reshape(-1)
    md = mods.astype(jnp.int32)
    vl = vals.astype(jnp.float32)
    out = _sc_kernel(fc, dg, md, vl, tab, pads[0], pads[1], pads[2])
    return out.reshape(N, D)


# speedup vs baseline: 1.0573x; 1.0025x over previous
import functools

import jax
import jax.numpy as jnp
from jax import lax
from jax.experimental import pallas as pl
from jax.experimental.pallas import tpu as pltpu
from jax.experimental.pallas import tpu_sc as plsc

N = 819200
L = 6
D = 64
PC = D // 2
ST = PC + 1
ROWS = 2 * 26 + 1000

NC = 2
NS = 16
NW = NC * NS
PER_W = N // NW
CP = 512
NCHUNK = PER_W // CP
NP = NCHUNK // 2
G = CP // 16

TAB_LEN = ROWS * ST
PAIR_LEN = 101 * ST
ZROW = 100 * ST
MASK_HI = -65536


def _unpack2(x):
    lo = plsc.bitcast(lax.shift_left(x, jnp.full_like(x, 16)), jnp.float32)
    hi = plsc.bitcast(lax.bitwise_and(x, jnp.full_like(x, MASK_HI)),
                      jnp.float32)
    return lo, hi


def _sc_body(fc_h, dg_h, md_h, vl_h, tab_h, p0_h, p1_h, p2_h, out_h,
             tabv, p0v, p1v, p2v, fcv, dgv, mdv, vlv, outv,
             ins0, ins1, outs0, outs1):
    c = lax.axis_index("c")
    s = lax.axis_index("s")
    wid = s * NC + c
    base0 = wid * PER_W

    pltpu.sync_copy(tab_h, tabv)
    pltpu.sync_copy(p0_h, p0v)
    pltpu.sync_copy(p1_h, p1v)
    pltpu.sync_copy(p2_h, p2v)

    iot = lax.iota(jnp.int32, 16)
    insems = (ins0, ins1)
    outsems = (outs0, outs1)

    def in_copies(ch, b):
        gb = base0 + ch * CP
        return (
            (fc_h.at[pl.ds(gb, CP)], fcv.at[pl.ds(b * CP, CP)]),
            (dg_h.at[pl.ds(gb * L, CP * L)], dgv.at[pl.ds(b * CP * L, CP * L)]),
            (md_h.at[pl.ds(gb, CP)], mdv.at[pl.ds(b * CP, CP)]),
            (vl_h.at[pl.ds(gb, CP)], vlv.at[pl.ds(b * CP, CP)]),
        )

    def start_in(ch, b):
        for src, dst in in_copies(ch, b):
            pltpu.async_copy(src, dst, insems[b])

    def drain_in(ch, b):
        for src, dst in in_copies(ch, b):
            pltpu.make_async_copy(src, dst, insems[b]).wait()

    def out_copy(ch, b):
        gb = base0 + ch * CP
        return (outv.at[pl.ds(b * CP * D, CP * D)],
                out_h.at[pl.ds(gb * D, CP * D)])

    def compute(b):
        ob_off = b * CP * D

        def group_body(g, carry2):
            t0 = b * CP + g * 16
            local = g * 16 + iot
            fc16 = fcv[pl.ds(t0, 16)]
            md16 = mdv[pl.ds(t0, 16)]
            vl16 = vlv[pl.ds(t0, 16)]
            t6 = local * L
            dgslice = dgv.at[pl.ds(b * CP * L, CP * L)]
            d0 = plsc.load_gather(dgslice, [t6])
            d1 = plsc.load_gather(dgslice, [t6 + 1])
            d2 = plsc.load_gather(dgslice, [t6 + 2])
            d3 = plsc.load_gather(dgslice, [t6 + 3])
            d4 = plsc.load_gather(dgslice, [t6 + 4])
            d5 = plsc.load_gather(dgslice, [t6 + 5])
            is_lab = md16 == 2
            zr = jnp.full_like(t6, ZROW)
            rb = fc16 * ST
            b01 = jnp.where(is_lab, zr, (d0 * 10 + d1) * ST)
            b23 = jnp.where(is_lab, zr, (d2 * 10 + d3) * ST)
            b45 = jnp.where(is_lab, zr, (d4 * 10 + d5) * ST)
            ob = local * D
            scale = jnp.where(is_lab, vl16, jnp.ones_like(vl16))
            scale2 = plsc.pack(scale, scale, format=plsc.PackFormat.INTERLEAVED)
            outsl = outv.at[pl.ds(ob_off, CP * D)]
            @plsc.parallel_loop(0, PC, unroll=4)
            def col_body(jp):
                jv = (iot + jp) & (PC - 1)
                bc = plsc.load_gather(tabv, [rb + jv])
                w1 = plsc.load_gather(p0v, [b01 + jv])
                w2 = plsc.load_gather(p1v, [b23 + jv])
                w3 = plsc.load_gather(p2v, [b45 + jv])
                bcb = plsc.bitcast(bc, jnp.bfloat16)
                w1b = plsc.bitcast(w1, jnp.bfloat16)
                w2b = plsc.bitcast(w2, jnp.bfloat16)
                w3b = plsc.bitcast(w3, jnp.bfloat16)
                o2 = bcb * scale2 + ((w1b + w2b) + w3b)
                oe, oo = plsc.unpack(o2, format=plsc.PackFormat.INTERLEAVED)
                obe = ob + (jv + jv)
                plsc.store_scatter(outsl, [obe], oe)
                plsc.store_scatter(outsl, [obe + 1], oo)
            return carry2

        lax.fori_loop(0, G, group_body, 0)

    start_in(0, 0)
    start_in(1, 1)

    def pair_body(jp, carry):
        for bbuf in (0, 1):
            ch = 2 * jp + bbuf
            drain_in(ch, bbuf)

            @pl.when(jp >= 1)
            def _wait_out():
                src, dst = out_copy(ch - 2, bbuf)
                pltpu.make_async_copy(src, dst, outsems[bbuf]).wait()

            compute(bbuf)
            src, dst = out_copy(ch, bbuf)
            pltpu.async_copy(src, dst, outsems[bbuf])

            @pl.when(jp < NP - 1)
            def _prefetch():
                start_in(ch + 2, bbuf)
        return carry

    lax.fori_loop(0, NP, pair_body, 0)
    for bbuf in (0, 1):
        src, dst = out_copy(NCHUNK - 2 + bbuf, bbuf)
        pltpu.make_async_copy(src, dst, outsems[bbuf]).wait()


@functools.partial(
    pl.kernel,
    out_type=jax.ShapeDtypeStruct((N * D,), jnp.float32),
    mesh=plsc.VectorSubcoreMesh(core_axis_name="c", subcore_axis_name="s"),
    compiler_params=pltpu.CompilerParams(needs_layout_passes=False),
    scratch_types=[
        pltpu.VMEM((TAB_LEN,), jnp.int32),
        pltpu.VMEM((PAIR_LEN,), jnp.int32),
        pltpu.VMEM((PAIR_LEN,), jnp.int32),
        pltpu.VMEM((PAIR_LEN,), jnp.int32),
        pltpu.VMEM((2 * CP,), jnp.int32),
        pltpu.VMEM((2 * CP * L,), jnp.int32),
        pltpu.VMEM((2 * CP,), jnp.int32),
        pltpu.VMEM((2 * CP,), jnp.float32),
        pltpu.VMEM((2 * CP * D,), jnp.float32),
        pltpu.SemaphoreType.DMA,
        pltpu.SemaphoreType.DMA,
        pltpu.SemaphoreType.DMA,
        pltpu.SemaphoreType.DMA,
    ],
)
def _sc_kernel(fc_h, dg_h, md_h, vl_h, tab_h, p0_h, p1_h, p2_h, out_h,
               tabv, p0v, p1v, p2v, fcv, dgv, mdv, vlv, outv,
               ins0, ins1, outs0, outs1):
    _sc_body(fc_h, dg_h, md_h, vl_h, tab_h, p0_h, p1_h, p2_h, out_h,
             tabv, p0v, p1v, p2v, fcv, dgv, mdv, vlv, outv,
             ins0, ins1, outs0, outs1)


def _pack_rows(t, pad_rows):
    tb = t.astype(jnp.bfloat16)
    u = lax.bitcast_convert_type(tb, jnp.uint16).astype(jnp.uint32)
    lo = u[:, 0::2]
    hi = u[:, 1::2]
    packed = (lo | (hi << 16)).astype(jnp.int32)
    return jnp.pad(packed, ((0, pad_rows), (0, 1))).reshape(-1)


def kernel(first_char, digits, mods, vals, kappa, first_level_w, integer_w):
    pos_w = 1.0 / (jnp.arange(L, dtype=jnp.float32) + 2.0) ** jnp.asarray(
        kappa, jnp.float32)
    pads = []
    for k in range(3):
        pk = (pos_w[2 * k] * integer_w[:, None, :]
              + pos_w[2 * k + 1] * integer_w[None, :, :]).reshape(100, D)
        pads.append(_pack_rows(pk, 1))
    tab = _pack_rows(first_level_w, 0)
    fc = first_char.astype(jnp.int32)
    dg = digits.astype(jnp.int32).reshape(-1)
    md = mods.astype(jnp.int32)
    vl = vals.astype(jnp.float32)
    out = _sc_kernel(fc, dg, md, vl, tab, pads[0], pads[1], pads[2])
    return out.reshape(N, D)


# speedup vs baseline: 1.0795x; 1.0210x over previous
import functools

import jax
import jax.numpy as jnp
from jax import lax
from jax.experimental import pallas as pl
from jax.experimental.pallas import tpu as pltpu
from jax.experimental.pallas import tpu_sc as plsc

N = 819200
L = 6
D = 64
PC = D // 2
ST = PC + 1
ROWS = 2 * 26 + 1000

NC = 2
NS = 16
NW = NC * NS
PER_W = N // NW
CP = 160
NCHUNK = PER_W // CP
NP = NCHUNK // 2
G = CP // 16

TAB_LEN = ROWS * ST
TRI_LEN = 1001 * ST
ZROW = 1000 * ST
MASK_HI = -65536


def _unpack2(x):
    lo = plsc.bitcast(lax.shift_left(x, jnp.full_like(x, 16)), jnp.float32)
    hi = plsc.bitcast(lax.bitwise_and(x, jnp.full_like(x, MASK_HI)),
                      jnp.float32)
    return lo, hi


def _sc_body(fc_h, dg_h, md_h, vl_h, tab_h, t0_h, t1_h, out_h,
             tabv, t0v, t1v, fcv, dgv, mdv, vlv, outv,
             ins0, ins1, outs0, outs1):
    c = lax.axis_index("c")
    s = lax.axis_index("s")
    wid = s * NC + c
    base0 = wid * PER_W

    pltpu.sync_copy(tab_h, tabv)
    pltpu.sync_copy(t0_h, t0v)
    pltpu.sync_copy(t1_h, t1v)

    iot = lax.iota(jnp.int32, 16)
    insems = (ins0, ins1)
    outsems = (outs0, outs1)

    def in_copies(ch, b):
        gb = base0 + ch * CP
        return (
            (fc_h.at[pl.ds(gb, CP)], fcv.at[pl.ds(b * CP, CP)]),
            (dg_h.at[pl.ds(gb * L, CP * L)], dgv.at[pl.ds(b * CP * L, CP * L)]),
            (md_h.at[pl.ds(gb, CP)], mdv.at[pl.ds(b * CP, CP)]),
            (vl_h.at[pl.ds(gb, CP)], vlv.at[pl.ds(b * CP, CP)]),
        )

    def start_in(ch, b):
        for src, dst in in_copies(ch, b):
            pltpu.async_copy(src, dst, insems[b])

    def drain_in(ch, b):
        for src, dst in in_copies(ch, b):
            pltpu.make_async_copy(src, dst, insems[b]).wait()

    def out_copy(ch, b):
        gb = base0 + ch * CP
        return (outv.at[pl.ds(b * CP * D, CP * D)],
                out_h.at[pl.ds(gb * D, CP * D)])

    def compute(b):
        ob_off = b * CP * D

        def group_body(g, carry2):
            t0 = b * CP + g * 16
            local = g * 16 + iot
            fc16 = fcv[pl.ds(t0, 16)]
            md16 = mdv[pl.ds(t0, 16)]
            vl16 = vlv[pl.ds(t0, 16)]
            t6 = local * L
            dgslice = dgv.at[pl.ds(b * CP * L, CP * L)]
            d0 = plsc.load_gather(dgslice, [t6])
            d1 = plsc.load_gather(dgslice, [t6 + 1])
            d2 = plsc.load_gather(dgslice, [t6 + 2])
            d3 = plsc.load_gather(dgslice, [t6 + 3])
            d4 = plsc.load_gather(dgslice, [t6 + 4])
            d5 = plsc.load_gather(dgslice, [t6 + 5])
            is_lab = md16 == 2
            zr = jnp.full_like(t6, ZROW)
            rb = fc16 * ST
            b012 = jnp.where(is_lab, zr, ((d0 * 10 + d1) * 10 + d2) * ST)
            b345 = jnp.where(is_lab, zr, ((d3 * 10 + d4) * 10 + d5) * ST)
            ob = local * D
            scale = jnp.where(is_lab, vl16, jnp.ones_like(vl16))
            scale2 = plsc.pack(scale, scale, format=plsc.PackFormat.INTERLEAVED)
            outsl = outv.at[pl.ds(ob_off, CP * D)]
            @plsc.parallel_loop(0, PC, unroll=4)
            def col_body(jp):
                jv = (iot + jp) & (PC - 1)
                bc = plsc.load_gather(tabv, [rb + jv])
                w1 = plsc.load_gather(t0v, [b012 + jv])
                w2 = plsc.load_gather(t1v, [b345 + jv])
                bcb = plsc.bitcast(bc, jnp.bfloat16)
                w1b = plsc.bitcast(w1, jnp.bfloat16)
                w2b = plsc.bitcast(w2, jnp.bfloat16)
                o2 = bcb * scale2 + (w1b + w2b)
                oe, oo = plsc.unpack(o2, format=plsc.PackFormat.INTERLEAVED)
                obe = ob + (jv + jv)
                plsc.store_scatter(outsl, [obe], oe)
                plsc.store_scatter(outsl, [obe + 1], oo)
            return carry2

        lax.fori_loop(0, G, group_body, 0)

    start_in(0, 0)
    start_in(1, 1)

    def pair_body(jp, carry):
        for bbuf in (0, 1):
            ch = 2 * jp + bbuf
            drain_in(ch, bbuf)

            @pl.when(jp >= 1)
            def _wait_out():
                src, dst = out_copy(ch - 2, bbuf)
                pltpu.make_async_copy(src, dst, outsems[bbuf]).wait()

            compute(bbuf)
            src, dst = out_copy(ch, bbuf)
            pltpu.async_copy(src, dst, outsems[bbuf])

            @pl.when(jp < NP - 1)
            def _prefetch():
                start_in(ch + 2, bbuf)
        return carry

    lax.fori_loop(0, NP, pair_body, 0)
    for bbuf in (0, 1):
        src, dst = out_copy(NCHUNK - 2 + bbuf, bbuf)
        pltpu.make_async_copy(src, dst, outsems[bbuf]).wait()


@functools.partial(
    pl.kernel,
    out_type=jax.ShapeDtypeStruct((N * D,), jnp.float32),
    mesh=plsc.VectorSubcoreMesh(core_axis_name="c", subcore_axis_name="s"),
    compiler_params=pltpu.CompilerParams(needs_layout_passes=False),
    scratch_types=[
        pltpu.VMEM((TAB_LEN,), jnp.int32),
        pltpu.VMEM((TRI_LEN,), jnp.int32),
        pltpu.VMEM((TRI_LEN,), jnp.int32),
        pltpu.VMEM((2 * CP,), jnp.int32),
        pltpu.VMEM((2 * CP * L,), jnp.int32),
        pltpu.VMEM((2 * CP,), jnp.int32),
        pltpu.VMEM((2 * CP,), jnp.float32),
        pltpu.VMEM((2 * CP * D,), jnp.float32),
        pltpu.SemaphoreType.DMA,
        pltpu.SemaphoreType.DMA,
        pltpu.SemaphoreType.DMA,
        pltpu.SemaphoreType.DMA,
    ],
)
def _sc_kernel(fc_h, dg_h, md_h, vl_h, tab_h, t0_h, t1_h, out_h,
               tabv, t0v, t1v, fcv, dgv, mdv, vlv, outv,
               ins0, ins1, outs0, outs1):
    _sc_body(fc_h, dg_h, md_h, vl_h, tab_h, t0_h, t1_h, out_h,
             tabv, t0v, t1v, fcv, dgv, mdv, vlv, outv,
             ins0, ins1, outs0, outs1)


def _pack_rows(t, pad_rows):
    tb = t.astype(jnp.bfloat16)
    u = lax.bitcast_convert_type(tb, jnp.uint16).astype(jnp.uint32)
    lo = u[:, 0::2]
    hi = u[:, 1::2]
    packed = (lo | (hi << 16)).astype(jnp.int32)
    return jnp.pad(packed, ((0, pad_rows), (0, 1))).reshape(-1)


def kernel(first_char, digits, mods, vals, kappa, first_level_w, integer_w):
    pos_w = 1.0 / (jnp.arange(L, dtype=jnp.float32) + 2.0) ** jnp.asarray(
        kappa, jnp.float32)
    pads = []
    for k in range(2):
        pk = (pos_w[3 * k] * integer_w[:, None, None, :]
              + pos_w[3 * k + 1] * integer_w[None, :, None, :]
              + pos_w[3 * k + 2] * integer_w[None, None, :, :]).reshape(1000, D)
        pads.append(_pack_rows(pk, 1))
    tab = _pack_rows(first_level_w, 0)
    fc = first_char.astype(jnp.int32)
    dg = digits.astype(jnp.int32).reshape(-1)
    md = mods.astype(jnp.int32)
    vl = vals.astype(jnp.float32)
    out = _sc_kernel(fc, dg, md, vl, tab, pads[0], pads[1])
    return out.reshape(N, D)


# speedup vs baseline: 1.0804x; 1.0009x over previous
import functools

import jax
import jax.numpy as jnp
from jax import lax
from jax.experimental import pallas as pl
from jax.experimental.pallas import tpu as pltpu
from jax.experimental.pallas import tpu_sc as plsc

N = 819200
L = 6
D = 64
PC = D // 2
ST = PC + 1
ROWS = 2 * 26 + 1000

NC = 2
NS = 16
NW = NC * NS
PER_W = N // NW
CP = 160
NCHUNK = PER_W // CP
NP = NCHUNK // 2
G = CP // 16

TAB_LEN = ROWS * ST
TRI_LEN = 1001 * ST
ZROW = 1000 * ST
MASK_HI = -65536


def _unpack2(x):
    lo = plsc.bitcast(lax.shift_left(x, jnp.full_like(x, 16)), jnp.float32)
    hi = plsc.bitcast(lax.bitwise_and(x, jnp.full_like(x, MASK_HI)),
                      jnp.float32)
    return lo, hi


def _sc_body(fc_h, dg_h, md_h, vl_h, tab_h, t0_h, t1_h, out_h,
             tabv, t0v, t1v, fcv, dgv, mdv, vlv, outv,
             ins0, ins1, outs0, outs1):
    c = lax.axis_index("c")
    s = lax.axis_index("s")
    wid = s * NC + c
    base0 = wid * PER_W

    pltpu.sync_copy(tab_h, tabv)
    pltpu.sync_copy(t0_h, t0v)
    pltpu.sync_copy(t1_h, t1v)

    iot = lax.iota(jnp.int32, 16)
    insems = (ins0, ins1)
    outsems = (outs0, outs1)

    def in_copies(ch, b):
        gb = base0 + ch * CP
        return (
            (fc_h.at[pl.ds(gb, CP)], fcv.at[pl.ds(b * CP, CP)]),
            (dg_h.at[pl.ds(gb * L, CP * L)], dgv.at[pl.ds(b * CP * L, CP * L)]),
            (md_h.at[pl.ds(gb, CP)], mdv.at[pl.ds(b * CP, CP)]),
            (vl_h.at[pl.ds(gb, CP)], vlv.at[pl.ds(b * CP, CP)]),
        )

    def start_in(ch, b):
        for src, dst in in_copies(ch, b):
            pltpu.async_copy(src, dst, insems[b])

    def drain_in(ch, b):
        for src, dst in in_copies(ch, b):
            pltpu.make_async_copy(src, dst, insems[b]).wait()

    def out_copy(ch, b):
        gb = base0 + ch * CP
        return (outv.at[pl.ds(b * CP * D, CP * D)],
                out_h.at[pl.ds(gb * D, CP * D)])

    def compute(b):
        ob_off = b * CP * D

        @plsc.parallel_loop(0, G, unroll=1)
        def group_body(g):
            t0 = b * CP + g * 16
            local = g * 16 + iot
            fc16 = fcv[pl.ds(t0, 16)]
            md16 = mdv[pl.ds(t0, 16)]
            vl16 = vlv[pl.ds(t0, 16)]
            t6 = local * L
            dgslice = dgv.at[pl.ds(b * CP * L, CP * L)]
            d0 = plsc.load_gather(dgslice, [t6])
            d1 = plsc.load_gather(dgslice, [t6 + 1])
            d2 = plsc.load_gather(dgslice, [t6 + 2])
            d3 = plsc.load_gather(dgslice, [t6 + 3])
            d4 = plsc.load_gather(dgslice, [t6 + 4])
            d5 = plsc.load_gather(dgslice, [t6 + 5])
            is_lab = md16 == 2
            zr = jnp.full_like(t6, ZROW)
            rb = fc16 * ST
            b012 = jnp.where(is_lab, zr, ((d0 * 10 + d1) * 10 + d2) * ST)
            b345 = jnp.where(is_lab, zr, ((d3 * 10 + d4) * 10 + d5) * ST)
            ob = local * D
            scale = jnp.where(is_lab, vl16, jnp.ones_like(vl16))
            scale2 = plsc.pack(scale, scale, format=plsc.PackFormat.INTERLEAVED)
            outsl = outv.at[pl.ds(ob_off, CP * D)]
            @plsc.parallel_loop(0, PC, unroll=4)
            def col_body(jp):
                jv = (iot + jp) & (PC - 1)
                bc = plsc.load_gather(tabv, [rb + jv])
                w1 = plsc.load_gather(t0v, [b012 + jv])
                w2 = plsc.load_gather(t1v, [b345 + jv])
                bcb = plsc.bitcast(bc, jnp.bfloat16)
                w1b = plsc.bitcast(w1, jnp.bfloat16)
                w2b = plsc.bitcast(w2, jnp.bfloat16)
                o2 = bcb * scale2 + (w1b + w2b)
                oe, oo = plsc.unpack(o2, format=plsc.PackFormat.INTERLEAVED)
                obe = ob + (jv + jv)
                plsc.store_scatter(outsl, [obe], oe)
                plsc.store_scatter(outsl, [obe + 1], oo)

    start_in(0, 0)
    start_in(1, 1)

    def pair_body(jp, carry):
        for bbuf in (0, 1):
            ch = 2 * jp + bbuf
            drain_in(ch, bbuf)

            @pl.when(jp >= 1)
            def _wait_out():
                src, dst = out_copy(ch - 2, bbuf)
                pltpu.make_async_copy(src, dst, outsems[bbuf]).wait()

            compute(bbuf)
            src, dst = out_copy(ch, bbuf)
            pltpu.async_copy(src, dst, outsems[bbuf])

            @pl.when(jp < NP - 1)
            def _prefetch():
                start_in(ch + 2, bbuf)
        return carry

    lax.fori_loop(0, NP, pair_body, 0)
    for bbuf in (0, 1):
        src, dst = out_copy(NCHUNK - 2 + bbuf, bbuf)
        pltpu.make_async_copy(src, dst, outsems[bbuf]).wait()


@functools.partial(
    pl.kernel,
    out_type=jax.ShapeDtypeStruct((N * D,), jnp.float32),
    mesh=plsc.VectorSubcoreMesh(core_axis_name="c", subcore_axis_name="s"),
    compiler_params=pltpu.CompilerParams(needs_layout_passes=False),
    scratch_types=[
        pltpu.VMEM((TAB_LEN,), jnp.int32),
        pltpu.VMEM((TRI_LEN,), jnp.int32),
        pltpu.VMEM((TRI_LEN,), jnp.int32),
        pltpu.VMEM((2 * CP,), jnp.int32),
        pltpu.VMEM((2 * CP * L,), jnp.int32),
        pltpu.VMEM((2 * CP,), jnp.int32),
        pltpu.VMEM((2 * CP,), jnp.float32),
        pltpu.VMEM((2 * CP * D,), jnp.float32),
        pltpu.SemaphoreType.DMA,
        pltpu.SemaphoreType.DMA,
        pltpu.SemaphoreType.DMA,
        pltpu.SemaphoreType.DMA,
    ],
)
def _sc_kernel(fc_h, dg_h, md_h, vl_h, tab_h, t0_h, t1_h, out_h,
               tabv, t0v, t1v, fcv, dgv, mdv, vlv, outv,
               ins0, ins1, outs0, outs1):
    _sc_body(fc_h, dg_h, md_h, vl_h, tab_h, t0_h, t1_h, out_h,
             tabv, t0v, t1v, fcv, dgv, mdv, vlv, outv,
             ins0, ins1, outs0, outs1)


def _pack_rows(t, pad_rows):
    tb = t.astype(jnp.bfloat16)
    u = lax.bitcast_convert_type(tb, jnp.uint16).astype(jnp.uint32)
    lo = u[:, 0::2]
    hi = u[:, 1::2]
    packed = (lo | (hi << 16)).astype(jnp.int32)
    return jnp.pad(packed, ((0, pad_rows), (0, 1))).reshape(-1)


def kernel(first_char, digits, mods, vals, kappa, first_level_w, integer_w):
    pos_w = 1.0 / (jnp.arange(L, dtype=jnp.float32) + 2.0) ** jnp.asarray(
        kappa, jnp.float32)
    pads = []
    for k in range(2):
        pk = (pos_w[3 * k] * integer_w[:, None, None, :]
              + pos_w[3 * k + 1] * integer_w[None, :, None, :]
              + pos_w[3 * k + 2] * integer_w[None, None, :, :]).reshape(1000, D)
        pads.append(_pack_rows(pk, 1))
    tab = _pack_rows(first_level_w, 0)
    fc = first_char.astype(jnp.int32)
    dg = digits.astype(jnp.int32).reshape(-1)
    md = mods.astype(jnp.int32)
    vl = vals.astype(jnp.float32)
    out = _sc_kernel(fc, dg, md, vl, tab, pads[0], pads[1])
    return out.reshape(N, D)


# speedup vs baseline: 1.0891x; 1.0080x over previous
import functools

import jax
import jax.numpy as jnp
from jax import lax
from jax.experimental import pallas as pl
from jax.experimental.pallas import tpu as pltpu
from jax.experimental.pallas import tpu_sc as plsc

N = 819200
L = 6
D = 64
PC = D // 2
ST = PC + 1
ROWS = 2 * 26 + 1000

NC = 2
NS = 16
NW = NC * NS
PER_W = N // NW
CP = 160
NCHUNK = PER_W // CP
NP = NCHUNK // 2
G = CP // 16

TAB_LEN = ROWS * ST
TRI_LEN = 1001 * ST
ZROW = 1000 * ST
MASK_HI = -65536


def _unpack2(x):
    lo = plsc.bitcast(lax.shift_left(x, jnp.full_like(x, 16)), jnp.float32)
    hi = plsc.bitcast(lax.bitwise_and(x, jnp.full_like(x, MASK_HI)),
                      jnp.float32)
    return lo, hi


def _sc_body(fc_h, dg_h, md_h, vl_h, tab_h, t0_h, t1_h, out_h,
             tabv, t0v, t1v, fcv, dgv, mdv, vlv, outv,
             ins0, ins1, outs0, outs1):
    c = lax.axis_index("c")
    s = lax.axis_index("s")
    wid = s * NC + c
    base0 = wid * PER_W

    pltpu.sync_copy(tab_h, tabv)
    pltpu.sync_copy(t0_h, t0v)
    pltpu.sync_copy(t1_h, t1v)

    iot = lax.iota(jnp.int32, 16)
    insems = (ins0, ins1)
    outsems = (outs0, outs1)

    def in_copies(ch, b):
        gb = base0 + ch * CP
        return (
            (fc_h.at[pl.ds(gb, CP)], fcv.at[pl.ds(b * CP, CP)]),
            (dg_h.at[pl.ds(gb * L, CP * L)], dgv.at[pl.ds(b * CP * L, CP * L)]),
            (md_h.at[pl.ds(gb, CP)], mdv.at[pl.ds(b * CP, CP)]),
            (vl_h.at[pl.ds(gb, CP)], vlv.at[pl.ds(b * CP, CP)]),
        )

    def start_in(ch, b):
        for src, dst in in_copies(ch, b):
            pltpu.async_copy(src, dst, insems[b])

    def drain_in(ch, b):
        for src, dst in in_copies(ch, b):
            pltpu.make_async_copy(src, dst, insems[b]).wait()

    def out_copy(ch, b):
        gb = base0 + ch * CP
        return (outv.at[pl.ds(b * CP * D, CP * D)],
                out_h.at[pl.ds(gb * D, CP * D)])

    def compute(b):
        ob_off = b * CP * D

        @plsc.parallel_loop(0, G, unroll=2)
        def group_body(g):
            t0 = b * CP + g * 16
            local = g * 16 + iot
            fc16 = fcv[pl.ds(t0, 16)]
            md16 = mdv[pl.ds(t0, 16)]
            vl16 = vlv[pl.ds(t0, 16)]
            t6 = local * L
            dgslice = dgv.at[pl.ds(b * CP * L, CP * L)]
            d0 = plsc.load_gather(dgslice, [t6])
            d1 = plsc.load_gather(dgslice, [t6 + 1])
            d2 = plsc.load_gather(dgslice, [t6 + 2])
            d3 = plsc.load_gather(dgslice, [t6 + 3])
            d4 = plsc.load_gather(dgslice, [t6 + 4])
            d5 = plsc.load_gather(dgslice, [t6 + 5])
            is_lab = md16 == 2
            zr = jnp.full_like(t6, ZROW)
            rb = fc16 * ST
            b012 = jnp.where(is_lab, zr, ((d0 * 10 + d1) * 10 + d2) * ST)
            b345 = jnp.where(is_lab, zr, ((d3 * 10 + d4) * 10 + d5) * ST)
            ob = local * D
            scale = jnp.where(is_lab, vl16, jnp.ones_like(vl16))
            scale2 = plsc.pack(scale, scale, format=plsc.PackFormat.INTERLEAVED)
            outsl = outv.at[pl.ds(ob_off, CP * D)]
            @plsc.parallel_loop(0, PC, unroll=4)
            def col_body(jp):
                jv = (iot + jp) & (PC - 1)
                bc = plsc.load_gather(tabv, [rb + jv])
                w1 = plsc.load_gather(t0v, [b012 + jv])
                w2 = plsc.load_gather(t1v, [b345 + jv])
                bcb = plsc.bitcast(bc, jnp.bfloat16)
                w1b = plsc.bitcast(w1, jnp.bfloat16)
                w2b = plsc.bitcast(w2, jnp.bfloat16)
                o2 = bcb * scale2 + (w1b + w2b)
                oe, oo = plsc.unpack(o2, format=plsc.PackFormat.INTERLEAVED)
                obe = ob + (jv + jv)
                plsc.store_scatter(outsl, [obe], oe)
                plsc.store_scatter(outsl, [obe + 1], oo)

    start_in(0, 0)
    start_in(1, 1)

    def pair_body(jp, carry):
        for bbuf in (0, 1):
            ch = 2 * jp + bbuf
            drain_in(ch, bbuf)

            @pl.when(jp >= 1)
            def _wait_out():
                src, dst = out_copy(ch - 2, bbuf)
                pltpu.make_async_copy(src, dst, outsems[bbuf]).wait()

            compute(bbuf)
            src, dst = out_copy(ch, bbuf)
            pltpu.async_copy(src, dst, outsems[bbuf])

            @pl.when(jp < NP - 1)
            def _prefetch():
                start_in(ch + 2, bbuf)
        return carry

    lax.fori_loop(0, NP, pair_body, 0)
    for bbuf in (0, 1):
        src, dst = out_copy(NCHUNK - 2 + bbuf, bbuf)
        pltpu.make_async_copy(src, dst, outsems[bbuf]).wait()


@functools.partial(
    pl.kernel,
    out_type=jax.ShapeDtypeStruct((N * D,), jnp.float32),
    mesh=plsc.VectorSubcoreMesh(core_axis_name="c", subcore_axis_name="s"),
    compiler_params=pltpu.CompilerParams(needs_layout_passes=False),
    scratch_types=[
        pltpu.VMEM((TAB_LEN,), jnp.int32),
        pltpu.VMEM((TRI_LEN,), jnp.int32),
        pltpu.VMEM((TRI_LEN,), jnp.int32),
        pltpu.VMEM((2 * CP,), jnp.int32),
        pltpu.VMEM((2 * CP * L,), jnp.int32),
        pltpu.VMEM((2 * CP,), jnp.int32),
        pltpu.VMEM((2 * CP,), jnp.float32),
        pltpu.VMEM((2 * CP * D,), jnp.float32),
        pltpu.SemaphoreType.DMA,
        pltpu.SemaphoreType.DMA,
        pltpu.SemaphoreType.DMA,
        pltpu.SemaphoreType.DMA,
    ],
)
def _sc_kernel(fc_h, dg_h, md_h, vl_h, tab_h, t0_h, t1_h, out_h,
               tabv, t0v, t1v, fcv, dgv, mdv, vlv, outv,
               ins0, ins1, outs0, outs1):
    _sc_body(fc_h, dg_h, md_h, vl_h, tab_h, t0_h, t1_h, out_h,
             tabv, t0v, t1v, fcv, dgv, mdv, vlv, outv,
             ins0, ins1, outs0, outs1)


def _pack_rows(t, pad_rows):
    tb = t.astype(jnp.bfloat16)
    u = lax.bitcast_convert_type(tb, jnp.uint16).astype(jnp.uint32)
    lo = u[:, 0::2]
    hi = u[:, 1::2]
    packed = (lo | (hi << 16)).astype(jnp.int32)
    return jnp.pad(packed, ((0, pad_rows), (0, 1))).reshape(-1)


def kernel(first_char, digits, mods, vals, kappa, first_level_w, integer_w):
    pos_w = 1.0 / (jnp.arange(L, dtype=jnp.float32) + 2.0) ** jnp.asarray(
        kappa, jnp.float32)
    pads = []
    for k in range(2):
        pk = (pos_w[3 * k] * integer_w[:, None, None, :]
              + pos_w[3 * k + 1] * integer_w[None, :, None, :]
              + pos_w[3 * k + 2] * integer_w[None, None, :, :]).reshape(1000, D)
        pads.append(_pack_rows(pk, 1))
    tab = _pack_rows(first_level_w, 0)
    fc = first_char.astype(jnp.int32)
    dg = digits.astype(jnp.int32).reshape(-1)
    md = mods.astype(jnp.int32)
    vl = vals.astype(jnp.float32)
    out = _sc_kernel(fc, dg, md, vl, tab, pads[0], pads[1])
    return out.reshape(N, D)
